# baseline (speedup 1.0000x reference)
import functools

import jax
import jax.numpy as jnp
from jax import lax
from jax.experimental import pallas as pl
from jax.experimental.pallas import tpu as pltpu
from jax.experimental.pallas import tpu_sc as plsc

_INTERPRET = False
_USE_SC = False

C = 768
M = 8
K = 8
CV = C // M
H = 32
W = 32
BS = 4
HW = H * W
N = BS * HW
T = BS * M
NPOS = K * HW

BLK = 512
GRID_MM = N // BLK

CHUNKP = 128
NCHUNKP = NPOS // CHUNKP


def _stage_a_body(q_ref, f_ref, wz_ref, bz_ref, woff_ref, boff_ref,
                  watt_ref, batt_ref, wp_ref, bp_ref,
                  off_ref, att_ref, wpo_ref):
    zq = jnp.dot(q_ref[...], wz_ref[...],
                 preferred_element_type=jnp.float32) + bz_ref[...]
    off_ref[...] = jnp.dot(zq, woff_ref[...],
                           preferred_element_type=jnp.float32) + boff_ref[...]
    att_ref[...] = jnp.dot(zq, watt_ref[...],
                           preferred_element_type=jnp.float32) + batt_ref[...]
    wpo_ref[...] = jnp.dot(f_ref[...], wp_ref[...],
                           preferred_element_type=jnp.float32) + bp_ref[...]


def _stage_a(q2, f2, wzT, bz2, woffT, boff2, wattT, batt2, wpT, bp2):
    row = lambda i: (i, 0)
    rep = lambda i: (0, 0)
    return pl.pallas_call(
        _stage_a_body,
        grid=(GRID_MM,),
        in_specs=[
            pl.BlockSpec((BLK, C), row),
            pl.BlockSpec((BLK, C), row),
            pl.BlockSpec((C, C), rep),
            pl.BlockSpec((1, C), rep),
            pl.BlockSpec((C, 2 * M * K), rep),
            pl.BlockSpec((1, 2 * M * K), rep),
            pl.BlockSpec((C, M * K), rep),
            pl.BlockSpec((1, M * K), rep),
            pl.BlockSpec((C, C), rep),
            pl.BlockSpec((1, C), rep),
        ],
        out_specs=[
            pl.BlockSpec((BLK, 2 * M * K), row),
            pl.BlockSpec((BLK, M * K), row),
            pl.BlockSpec((BLK, C), row),
        ],
        out_shape=[
            jax.ShapeDtypeStruct((N, 2 * M * K), jnp.float32),
            jax.ShapeDtypeStruct((N, M * K), jnp.float32),
            jax.ShapeDtypeStruct((N, C), jnp.float32),
        ],
        interpret=_INTERPRET,
    )(q2, f2, wzT, bz2, woffT, boff2, wattT, batt2, wpT, bp2)


def _stage_b_body(offx_ref, offy_ref, phix_ref, phiy_ref,
                  i00, i10, i01, i11, c00, c10, c01, c11):
    ix = (phix_ref[...] + offx_ref[...]) * (W / (W - 1.0)) - 0.5
    iy = (phiy_ref[...] + offy_ref[...]) * (H / (H - 1.0)) - 0.5
    x0 = jnp.floor(ix)
    y0 = jnp.floor(iy)
    wx1 = ix - x0
    wy1 = iy - y0
    wx0 = 1.0 - wx1
    wy0 = 1.0 - wy1
    x1 = x0 + 1.0
    y1 = y0 + 1.0

    def tap(xf, yf, wgt, iref, cref):
        valid = ((xf >= 0) & (xf <= W - 1) & (yf >= 0) & (yf <= H - 1))
        xc = jnp.clip(xf, 0.0, W - 1.0)
        yc = jnp.clip(yf, 0.0, H - 1.0)
        iref[...] = (yc * W + xc).astype(jnp.int32)
        cref[...] = wgt * valid.astype(jnp.float32)

    tap(x0, y0, wx0 * wy0, i00, c00)
    tap(x1, y0, wx1 * wy0, i10, c10)
    tap(x0, y1, wx0 * wy1, i01, c01)
    tap(x1, y1, wx1 * wy1, i11, c11)


def _stage_b(offx, offy, phix, phiy):
    TB = 4
    blk = lambda i: (i, 0, 0)
    out_spec = pl.BlockSpec((TB, K, HW), blk)
    return pl.pallas_call(
        _stage_b_body,
        grid=(T // TB,),
        in_specs=[
            pl.BlockSpec((TB, K, HW), blk),
            pl.BlockSpec((TB, K, HW), blk),
            pl.BlockSpec((TB, 1, HW), blk),
            pl.BlockSpec((TB, 1, HW), blk),
        ],
        out_specs=[out_spec] * 8,
        out_shape=[jax.ShapeDtypeStruct((T, K, HW), jnp.int32)] * 4
        + [jax.ShapeDtypeStruct((T, K, HW), jnp.float32)] * 4,
        interpret=_INTERPRET,
    )(offx, offy, phix, phiy)


def _sc_sample(tables, idx, cf):
    mesh = plsc.VectorSubcoreMesh(core_axis_name="c", subcore_axis_name="s")

    @functools.partial(
        pl.kernel,
        out_type=jax.ShapeDtypeStruct((T, NPOS, CV), jnp.float32),
        mesh=mesh,
        scratch_types=[
            pltpu.VMEM((HW, CV), jnp.float32),
            pltpu.VMEM((CHUNKP, 4), jnp.int32),
            pltpu.VMEM((CHUNKP, 4), jnp.float32),
            pltpu.VMEM((CHUNKP, CV), jnp.float32),
        ],
    )
    def samp(tab_hbm, idx_hbm, cf_hbm, out_hbm, tab_v, idx_v, cf_v, out_v):
        t = lax.axis_index("s") * 2 + lax.axis_index("c")
        pltpu.sync_copy(tab_hbm.at[t], tab_v)

        def chunk_body(ci, carry):
            base = ci * CHUNKP
            pltpu.sync_copy(idx_hbm.at[t, pl.ds(base, CHUNKP)], idx_v)
            pltpu.sync_copy(cf_hbm.at[t, pl.ds(base, CHUNKP)], cf_v)

            def n_body(n, carry2):
                accs = [jnp.zeros((16,), jnp.float32) for _ in range(CV // 16)]
                for j in range(4):
                    lin = idx_v[n, j]
                    wgt = cf_v[n, j]
                    for c in range(CV // 16):
                        accs[c] = accs[c] + wgt * tab_v[lin, pl.ds(c * 16, 16)]
                for c in range(CV // 16):
                    out_v[n, pl.ds(c * 16, 16)] = accs[c]
                return carry2

            lax.fori_loop(0, CHUNKP, n_body, 0)
            pltpu.sync_copy(out_v, out_hbm.at[t, pl.ds(base, CHUNKP)])
            return carry

        lax.fori_loop(0, NCHUNKP, chunk_body, 0)

    return samp(tables, idx, cf)


def _jnp_sample(tables, idx, cf):
    rows = jax.vmap(lambda tab, ii: tab[ii])(tables, idx)
    return jnp.einsum('tpjc,tpj->tpc', rows, cf)


def _stage_d_body(s_ref, a_ref, ht_ref, g_ref, o_ref):
    a = a_ref[...]
    amax = jnp.max(a, axis=1, keepdims=True)
    e = jnp.exp(a - amax)
    attn = e / jnp.sum(e, axis=1, keepdims=True)
    ab = jnp.dot(attn, ht_ref[...],
                 preferred_element_type=jnp.float32)
    p = s_ref[...] * ab
    o_ref[...] = jnp.dot(p, g_ref[...],
                         preferred_element_type=jnp.float32)


def _stage_d(stacked2, attn2, htile, gsum):
    row = lambda i: (i, 0)
    rep = lambda i: (0, 0)
    return pl.pallas_call(
        _stage_d_body,
        grid=(T,),
        in_specs=[
            pl.BlockSpec((HW, C), row),
            pl.BlockSpec((HW, K), row),
            pl.BlockSpec((K, C), rep),
            pl.BlockSpec((C, CV), rep),
        ],
        out_specs=pl.BlockSpec((HW, CV), row),
        out_shape=jax.ShapeDtypeStruct((T * HW, CV), jnp.float32),
        interpret=_INTERPRET,
    )(stacked2, attn2, htile, gsum)


def _stage_e_body(x_ref, w_ref, b_ref, o_ref):
    o_ref[...] = jnp.dot(x_ref[...], w_ref[...],
                         preferred_element_type=jnp.float32) + b_ref[...]


def _stage_e(x2, wmT, bm2):
    row = lambda i: (i, 0)
    rep = lambda i: (0, 0)
    return pl.pallas_call(
        _stage_e_body,
        grid=(GRID_MM,),
        in_specs=[
            pl.BlockSpec((BLK, C), row),
            pl.BlockSpec((C, C), rep),
            pl.BlockSpec((1, C), rep),
        ],
        out_specs=pl.BlockSpec((BLK, C), row),
        out_shape=jax.ShapeDtypeStruct((N, C), jnp.float32),
        interpret=_INTERPRET,
    )(x2, wmT, bm2)


def kernel(q, features, ref, Wz, bz, Woff, boff, Watt, batt, Wp, bp, Wm, bm):
    q2 = q.reshape(N, C)
    f2 = features[0].reshape(N, C)

    off_raw, att_raw, wp2 = _stage_a(
        q2, f2,
        Wz.T, bz.reshape(1, C),
        Woff.T, boff.reshape(1, 2 * M * K),
        Watt.T, batt.reshape(1, M * K),
        Wp.T, bp.reshape(1, C))

    off6 = off_raw.reshape(BS, H, W, M, K, 2)
    offx = jnp.transpose(off6[..., 0], (0, 3, 4, 2, 1)).reshape(T, K, HW)
    offy = jnp.transpose(off6[..., 1], (0, 3, 4, 2, 1)).reshape(T, K, HW)
    phix = jnp.tile(jnp.transpose(ref[..., 0], (0, 2, 1)).reshape(BS, 1, HW)
                    * (W - 1.0), (M, 1, 1))
    phiy = jnp.tile(jnp.transpose(ref[..., 1], (0, 2, 1)).reshape(BS, 1, HW)
                    * (H - 1.0), (M, 1, 1))

    i00, i10, i01, i11, c00, c10, c01, c11 = _stage_b(offx, offy, phix, phiy)

    idx = jnp.stack([i00, i10, i01, i11], axis=3).reshape(T, NPOS, 4)
    cf = jnp.stack([c00, c10, c01, c11], axis=3).reshape(T, NPOS, 4)

    tables = wp2.reshape(BS, HW, M, CV).transpose(0, 2, 1, 3).reshape(T, HW, CV)

    if _USE_SC:
        samp = _sc_sample(tables, idx, cf)
    else:
        samp = _jnp_sample(tables, idx, cf)

    stacked2 = samp.reshape(T * HW, C)
    attn2 = att_raw.reshape(BS, HW, M, K).transpose(0, 2, 1, 3).reshape(T * HW, K)

    ii = jnp.arange(C, dtype=jnp.int32)
    htile = (ii[None, :] % K == jnp.arange(K, dtype=jnp.int32)[:, None]
             ).astype(jnp.float32)
    gsum = (ii[:, None] // K == jnp.arange(CV, dtype=jnp.int32)[None, :]
            ).astype(jnp.float32)

    att_out2 = _stage_d(stacked2, attn2, htile, gsum)

    att_out = att_out2.reshape(BS, M, HW, CV).transpose(0, 2, 1, 3).reshape(N, C)
    final = _stage_e(att_out, Wm.T, bm.reshape(1, C))
    return final.reshape(BS, H, W, C)


# speedup vs baseline: 27.7422x; 27.7422x over previous
import functools

import jax
import jax.numpy as jnp
from jax import lax
from jax.experimental import pallas as pl
from jax.experimental.pallas import tpu as pltpu
from jax.experimental.pallas import tpu_sc as plsc

_INTERPRET = False
_USE_SC = True

C = 768
M = 8
K = 8
CV = C // M
H = 32
W = 32
BS = 4
HW = H * W
N = BS * HW
T = BS * M
NPOS = K * HW

BLK = 512
GRID_MM = N // BLK

CHUNKP = 128
NCHUNKP = NPOS // CHUNKP


def _stage_a_body(q_ref, f_ref, wz_ref, bz_ref, woff_ref, boff_ref,
                  watt_ref, batt_ref, wp_ref, bp_ref,
                  off_ref, att_ref, wpo_ref):
    zq = jnp.dot(q_ref[...], wz_ref[...],
                 preferred_element_type=jnp.float32) + bz_ref[...]
    off_ref[...] = jnp.dot(zq, woff_ref[...],
                           preferred_element_type=jnp.float32) + boff_ref[...]
    att_ref[...] = jnp.dot(zq, watt_ref[...],
                           preferred_element_type=jnp.float32) + batt_ref[...]
    wpo_ref[...] = jnp.dot(f_ref[...], wp_ref[...],
                           preferred_element_type=jnp.float32) + bp_ref[...]


def _stage_a(q2, f2, wzT, bz2, woffT, boff2, wattT, batt2, wpT, bp2):
    row = lambda i: (i, 0)
    rep = lambda i: (0, 0)
    return pl.pallas_call(
        _stage_a_body,
        grid=(GRID_MM,),
        in_specs=[
            pl.BlockSpec((BLK, C), row),
            pl.BlockSpec((BLK, C), row),
            pl.BlockSpec((C, C), rep),
            pl.BlockSpec((1, C), rep),
            pl.BlockSpec((C, 2 * M * K), rep),
            pl.BlockSpec((1, 2 * M * K), rep),
            pl.BlockSpec((C, M * K), rep),
            pl.BlockSpec((1, M * K), rep),
            pl.BlockSpec((C, C), rep),
            pl.BlockSpec((1, C), rep),
        ],
        out_specs=[
            pl.BlockSpec((BLK, 2 * M * K), row),
            pl.BlockSpec((BLK, M * K), row),
            pl.BlockSpec((BLK, C), row),
        ],
        out_shape=[
            jax.ShapeDtypeStruct((N, 2 * M * K), jnp.float32),
            jax.ShapeDtypeStruct((N, M * K), jnp.float32),
            jax.ShapeDtypeStruct((N, C), jnp.float32),
        ],
        interpret=_INTERPRET,
    )(q2, f2, wzT, bz2, woffT, boff2, wattT, batt2, wpT, bp2)


def _stage_b_body(offx_ref, offy_ref, phix_ref, phiy_ref,
                  i00, i10, i01, i11, c00, c10, c01, c11):
    ix = (phix_ref[...] + offx_ref[...]) * (W / (W - 1.0)) - 0.5
    iy = (phiy_ref[...] + offy_ref[...]) * (H / (H - 1.0)) - 0.5
    x0 = jnp.floor(ix)
    y0 = jnp.floor(iy)
    wx1 = ix - x0
    wy1 = iy - y0
    wx0 = 1.0 - wx1
    wy0 = 1.0 - wy1
    x1 = x0 + 1.0
    y1 = y0 + 1.0

    def tap(xf, yf, wgt, iref, cref):
        valid = ((xf >= 0) & (xf <= W - 1) & (yf >= 0) & (yf <= H - 1))
        xc = jnp.clip(xf, 0.0, W - 1.0)
        yc = jnp.clip(yf, 0.0, H - 1.0)
        iref[...] = (yc * W + xc).astype(jnp.int32)
        cref[...] = wgt * valid.astype(jnp.float32)

    tap(x0, y0, wx0 * wy0, i00, c00)
    tap(x1, y0, wx1 * wy0, i10, c10)
    tap(x0, y1, wx0 * wy1, i01, c01)
    tap(x1, y1, wx1 * wy1, i11, c11)


def _stage_b(offx, offy, phix, phiy):
    TB = 4
    blk = lambda i: (i, 0, 0)
    out_spec = pl.BlockSpec((TB, K, HW), blk)
    return pl.pallas_call(
        _stage_b_body,
        grid=(T // TB,),
        in_specs=[
            pl.BlockSpec((TB, K, HW), blk),
            pl.BlockSpec((TB, K, HW), blk),
            pl.BlockSpec((TB, 1, HW), blk),
            pl.BlockSpec((TB, 1, HW), blk),
        ],
        out_specs=[out_spec] * 8,
        out_shape=[jax.ShapeDtypeStruct((T, K, HW), jnp.int32)] * 4
        + [jax.ShapeDtypeStruct((T, K, HW), jnp.float32)] * 4,
        interpret=_INTERPRET,
    )(offx, offy, phix, phiy)


def _sc_sample(tables, idx, cf):
    mesh = plsc.VectorSubcoreMesh(core_axis_name="c", subcore_axis_name="s")

    @functools.partial(
        pl.kernel,
        out_type=jax.ShapeDtypeStruct((T, NPOS * CV), jnp.float32),
        mesh=mesh,
        scratch_types=[
            pltpu.VMEM((HW * CV,), jnp.float32),
            pltpu.VMEM((CHUNKP * 4,), jnp.int32),
            pltpu.VMEM((CHUNKP * 4,), jnp.float32),
            pltpu.VMEM((CHUNKP * CV,), jnp.float32),
        ],
    )
    def samp(tab_hbm, idx_hbm, cf_hbm, out_hbm, tab_v, idx_v, cf_v, out_v):
        t = lax.axis_index("s") * 2 + lax.axis_index("c")
        pltpu.sync_copy(tab_hbm.at[t], tab_v)

        def chunk_body(ci, carry):
            base = ci * CHUNKP
            pltpu.sync_copy(idx_hbm.at[t, pl.ds(base * 4, CHUNKP * 4)], idx_v)
            pltpu.sync_copy(cf_hbm.at[t, pl.ds(base * 4, CHUNKP * 4)], cf_v)

            def g_body(g, carry2):
                iv = idx_v[pl.ds(g * 16, 16)]
                wv = cf_v[pl.ds(g * 16, 16)]
                for p in range(4):
                    accs = [jnp.zeros((16,), jnp.float32)
                            for _ in range(CV // 16)]
                    for j in range(4):
                        lin = iv[p * 4 + j]
                        wgt = wv[p * 4 + j]
                        for c in range(CV // 16):
                            accs[c] = accs[c] + wgt * tab_v[
                                pl.ds(lin * CV + c * 16, 16)]
                    for c in range(CV // 16):
                        out_v[pl.ds((g * 4 + p) * CV + c * 16, 16)] = accs[c]
                return carry2

            lax.fori_loop(0, CHUNKP // 4, g_body, 0)
            pltpu.sync_copy(out_v, out_hbm.at[t, pl.ds(base * CV, CHUNKP * CV)])
            return carry

        lax.fori_loop(0, NCHUNKP, chunk_body, 0)

    return samp(tables, idx, cf)


def _jnp_sample(tables, idx, cf):
    idx4 = idx.reshape(T, NPOS, 4)
    cf4 = cf.reshape(T, NPOS, 4)
    tab3 = tables.reshape(T, HW, CV)
    rows = jax.vmap(lambda tab, ii: tab[ii])(tab3, idx4)
    return jnp.einsum('tpjc,tpj->tpc', rows, cf4).reshape(T, NPOS * CV)


def _stage_d_body(s_ref, a_ref, ht_ref, g_ref, o_ref):
    a = a_ref[...]
    amax = jnp.max(a, axis=1, keepdims=True)
    e = jnp.exp(a - amax)
    attn = e / jnp.sum(e, axis=1, keepdims=True)
    ab = jnp.dot(attn, ht_ref[...],
                 preferred_element_type=jnp.float32)
    p = s_ref[...] * ab
    o_ref[...] = jnp.dot(p, g_ref[...],
                         preferred_element_type=jnp.float32)


def _stage_d(stacked2, attn2, htile, gsum):
    row = lambda i: (i, 0)
    rep = lambda i: (0, 0)
    return pl.pallas_call(
        _stage_d_body,
        grid=(T,),
        in_specs=[
            pl.BlockSpec((HW, C), row),
            pl.BlockSpec((HW, K), row),
            pl.BlockSpec((K, C), rep),
            pl.BlockSpec((C, CV), rep),
        ],
        out_specs=pl.BlockSpec((HW, CV), row),
        out_shape=jax.ShapeDtypeStruct((T * HW, CV), jnp.float32),
        interpret=_INTERPRET,
    )(stacked2, attn2, htile, gsum)


def _stage_e_body(x_ref, w_ref, b_ref, o_ref):
    o_ref[...] = jnp.dot(x_ref[...], w_ref[...],
                         preferred_element_type=jnp.float32) + b_ref[...]


def _stage_e(x2, wmT, bm2):
    row = lambda i: (i, 0)
    rep = lambda i: (0, 0)
    return pl.pallas_call(
        _stage_e_body,
        grid=(GRID_MM,),
        in_specs=[
            pl.BlockSpec((BLK, C), row),
            pl.BlockSpec((C, C), rep),
            pl.BlockSpec((1, C), rep),
        ],
        out_specs=pl.BlockSpec((BLK, C), row),
        out_shape=jax.ShapeDtypeStruct((N, C), jnp.float32),
        interpret=_INTERPRET,
    )(x2, wmT, bm2)


def kernel(q, features, ref, Wz, bz, Woff, boff, Watt, batt, Wp, bp, Wm, bm):
    q2 = q.reshape(N, C)
    f2 = features[0].reshape(N, C)

    off_raw, att_raw, wp2 = _stage_a(
        q2, f2,
        Wz.T, bz.reshape(1, C),
        Woff.T, boff.reshape(1, 2 * M * K),
        Watt.T, batt.reshape(1, M * K),
        Wp.T, bp.reshape(1, C))

    off6 = off_raw.reshape(BS, H, W, M, K, 2)
    offx = jnp.transpose(off6[..., 0], (0, 3, 4, 2, 1)).reshape(T, K, HW)
    offy = jnp.transpose(off6[..., 1], (0, 3, 4, 2, 1)).reshape(T, K, HW)
    phix = jnp.tile(jnp.transpose(ref[..., 0], (0, 2, 1)).reshape(BS, 1, HW)
                    * (W - 1.0), (M, 1, 1))
    phiy = jnp.tile(jnp.transpose(ref[..., 1], (0, 2, 1)).reshape(BS, 1, HW)
                    * (H - 1.0), (M, 1, 1))

    i00, i10, i01, i11, c00, c10, c01, c11 = _stage_b(offx, offy, phix, phiy)

    idx = jnp.stack([i00, i10, i01, i11], axis=3).reshape(T, NPOS * 4)
    cf = jnp.stack([c00, c10, c01, c11], axis=3).reshape(T, NPOS * 4)

    tables = wp2.reshape(BS, HW, M, CV).transpose(0, 2, 1, 3).reshape(T, HW * CV)

    if _USE_SC:
        samp = _sc_sample(tables, idx, cf)
    else:
        samp = _jnp_sample(tables, idx, cf)

    stacked2 = samp.reshape(T * HW, C)
    attn2 = att_raw.reshape(BS, HW, M, K).transpose(0, 2, 1, 3).reshape(T * HW, K)

    ii = jnp.arange(C, dtype=jnp.int32)
    htile = (ii[None, :] % K == jnp.arange(K, dtype=jnp.int32)[:, None]
             ).astype(jnp.float32)
    gsum = (ii[:, None] // K == jnp.arange(CV, dtype=jnp.int32)[None, :]
            ).astype(jnp.float32)

    att_out2 = _stage_d(stacked2, attn2, htile, gsum)

    att_out = att_out2.reshape(BS, M, HW, CV).transpose(0, 2, 1, 3).reshape(N, C)
    final = _stage_e(att_out, Wm.T, bm.reshape(1, C))
    return final.reshape(BS, H, W, C)
